# baseline (speedup 1.0000x reference)
import functools

import jax
import jax.numpy as jnp
from jax import lax
from jax.experimental import pallas as pl
from jax.experimental.pallas import tpu as pltpu
from jax.experimental.pallas import tpu_sc as plsc

N = 10000
E = 160000
G = 64
NP = 10240
ROWS_PER_TILE = NP // 16
NW = 32
CH = 128
NCHUNK = 40
EPT = CH * NCHUNK
EP = NW * EPT

_mesh = plsc.VectorSubcoreMesh(core_axis_name="c", subcore_axis_name="s")


def _zero_rows(zbuf, n_rows, d):
    def body(i, _):
        for k in range(d // 16):
            zbuf[i, pl.ds(16 * k, 16)] = jnp.zeros((16,), jnp.float32)
        return 0
    lax.fori_loop(0, n_rows, body, 0)


def _deg_body(col_hbm, ew_hbm, out_hbm, col_v, ew_v, zbuf, deg_sh):
    cid = lax.axis_index("c")
    sid = lax.axis_index("s")
    wid = cid * 16 + sid
    pltpu.sync_copy(col_hbm.at[wid], col_v)
    pltpu.sync_copy(ew_hbm.at[wid], ew_v)

    def zb(i, _):
        zbuf[pl.ds(16 * i, 16)] = jnp.zeros((16,), jnp.float32)
        return 0
    lax.fori_loop(0, ROWS_PER_TILE // 16, zb, 0)
    pltpu.sync_copy(zbuf, deg_sh.at[pl.ds(sid * ROWS_PER_TILE, ROWS_PER_TILE)])
    plsc.subcore_barrier()

    def chunk(j, _):
        pltpu.sync_copy(ew_v.at[j], deg_sh.at[col_v.at[j]], add=True)
        return 0
    lax.fori_loop(0, NCHUNK, chunk, 0)
    plsc.subcore_barrier()
    pltpu.sync_copy(deg_sh.at[pl.ds(sid * ROWS_PER_TILE, ROWS_PER_TILE)],
                    out_hbm.at[cid, pl.ds(sid * ROWS_PER_TILE, ROWS_PER_TILE)])


@functools.partial(
    pl.kernel,
    out_type=jax.ShapeDtypeStruct((2, NP), jnp.float32),
    mesh=_mesh,
    scratch_types=[
        pltpu.VMEM((NCHUNK, CH), jnp.int32),
        pltpu.VMEM((NCHUNK, CH), jnp.float32),
        pltpu.VMEM((ROWS_PER_TILE,), jnp.float32),
        pltpu.VMEM_SHARED((NP,), jnp.float32),
    ],
)
def _deg_call(col_hbm, ew_hbm, out_hbm, col_v, ew_v, zbuf, deg_sh):
    _deg_body(col_hbm, ew_hbm, out_hbm, col_v, ew_v, zbuf, deg_sh)


NBUF = 2
NROW_TILE = N // 16


def _agg_body(d, ch, nchunk, row_hbm, col_hbm, ew_hbm, y_hbm, out_hbm,
              row_v, col_v, ew_v, gbufs, acc_sh, gsems):
    cid = lax.axis_index("c")
    sid = lax.axis_index("s")
    wid = cid * 16 + sid
    pltpu.sync_copy(row_hbm.at[wid], row_v)
    pltpu.sync_copy(col_hbm.at[wid], col_v)
    pltpu.sync_copy(ew_hbm.at[wid], ew_v)
    _zero_rows(gbufs[0], ch, d)
    for k in range(NROW_TILE // ch):
        pltpu.sync_copy(gbufs[0], acc_sh.at[pl.ds(sid * NROW_TILE + k * ch, ch)])
    rem = NROW_TILE % ch
    if rem:
        pltpu.sync_copy(
            gbufs[0].at[pl.ds(0, rem)],
            acc_sh.at[pl.ds(sid * NROW_TILE + (NROW_TILE // ch) * ch, rem)])
    plsc.subcore_barrier()

    def scale_chunk(j, gbuf):
        for g in range(ch // 16):
            ew16 = ew_v[j * (ch // 16) + g]
            for r in range(16):
                s = lax.gather(
                    ew16, jnp.full((16, 1), r, jnp.int32),
                    lax.GatherDimensionNumbers(
                        offset_dims=(), collapsed_slice_dims=(0,),
                        start_index_map=(0,)),
                    (1,), mode=lax.GatherScatterMode.PROMISE_IN_BOUNDS)
                e = g * 16 + r
                for k in range(d // 16):
                    gbuf[e, pl.ds(16 * k, 16)] = gbuf[e, pl.ds(16 * k, 16)] * s

    def outer(j, _):
        pltpu.async_copy(y_hbm.at[row_v.at[j]], gbufs[0], gsems[0]).wait()
        scale_chunk(j, gbufs[0])
        pltpu.sync_copy(gbufs[0], acc_sh.at[col_v.at[j]], add=True)
        return 0
    lax.fori_loop(0, nchunk, outer, 0)
    plsc.subcore_barrier()
    pltpu.sync_copy(acc_sh.at[pl.ds(sid * NROW_TILE, NROW_TILE)],
                    out_hbm.at[cid, pl.ds(sid * NROW_TILE, NROW_TILE)])


def _make_agg(d, ch):
    nchunk = EPT // ch
    @functools.partial(
        pl.kernel,
        out_type=jax.ShapeDtypeStruct((2, N, d), jnp.float32),
        mesh=_mesh,
        scratch_types=[
            pltpu.VMEM((nchunk, ch), jnp.int32),
            pltpu.VMEM((nchunk, ch), jnp.int32),
            pltpu.VMEM((EPT // 16, 16), jnp.float32),
        ] + [pltpu.VMEM((ch, d), jnp.float32)] * 1 + [
            pltpu.VMEM_SHARED((N, d), jnp.float32),
        ] + [pltpu.SemaphoreType.DMA] * 1,
        compiler_params=pltpu.CompilerParams(use_tc_tiling_on_sc=False),
        name=f"gcn_agg_d{d}",
    )
    def agg(row_hbm, col_hbm, ew_hbm, y_hbm, out_hbm,
            row_v, col_v, ew_v, g0, acc_sh, gs0):
        _agg_body(d, ch, nchunk, row_hbm, col_hbm, ew_hbm, y_hbm, out_hbm,
                  row_v, col_v, ew_v, (g0,), acc_sh, (gs0,))
    return agg


_agg_128 = _make_agg(128, 160)
_agg_64 = _make_agg(64, 160)
_agg_32 = _make_agg(32, 160)


def _tc1_body(parts_ref, x_ref, w_ref, dinv_ref, y_ref):
    p = parts_ref[...]
    deg = p[0, :N] + p[1, :N] + 1.0
    dinv = jnp.where(deg > 0, lax.rsqrt(deg), 0.0)
    t = jnp.dot(x_ref[...], w_ref[...], preferred_element_type=jnp.float32)
    dinv_ref[...] = dinv
    y_ref[...] = dinv * t


def _tc1(parts, x, w):
    return pl.pallas_call(
        _tc1_body,
        out_shape=(jax.ShapeDtypeStruct((N, 1), jnp.float32),
                   jax.ShapeDtypeStruct((N, w.shape[1]), jnp.float32)),
    )(parts, x, w)


def _tc_mid_body(parts_ref, y_ref, dinv_ref, b_ref, g_ref, be_ref, w_ref, ynext_ref):
    p = parts_ref[...]
    agg = p[0, :N] + p[1, :N]
    dinv = dinv_ref[...]
    z = jax.nn.relu(dinv * (agg + y_ref[...]) + b_ref[...])
    m = jnp.mean(z, axis=0, keepdims=True)
    v = jnp.mean((z - m) ** 2, axis=0, keepdims=True)
    h = g_ref[...] * (z - m) * lax.rsqrt(v + 1e-5) + be_ref[...]
    ynext_ref[...] = dinv * jnp.dot(h, w_ref[...], preferred_element_type=jnp.float32)


def _tc_mid(parts, y, dinv, b, g, be, w):
    return pl.pallas_call(
        _tc_mid_body,
        out_shape=jax.ShapeDtypeStruct((N, w.shape[1]), jnp.float32),
    )(parts, y, dinv, b.reshape(1, -1), g.reshape(1, -1), be.reshape(1, -1), w)


def _tc_fin_body(parts_ref, y_ref, dinv_ref, b_ref, g_ref, be_ref, batch_ref,
                 fw1_ref, fb1_ref, fw2_ref, fb2_ref, fw3_ref, fb3_ref, out_ref):
    p = parts_ref[...]
    agg = p[0, :N] + p[1, :N]
    dinv = dinv_ref[...]
    z = jax.nn.relu(dinv * (agg + y_ref[...]) + b_ref[...])
    m = jnp.mean(z, axis=0, keepdims=True)
    v = jnp.mean((z - m) ** 2, axis=0, keepdims=True)
    h = g_ref[...] * (z - m) * lax.rsqrt(v + 1e-5) + be_ref[...]
    oh = (batch_ref[...] == lax.broadcasted_iota(jnp.int32, (N, G), 1))
    pooled = lax.dot_general(oh.astype(jnp.float32), h,
                             (((0,), (0,)), ((), ())),
                             precision=lax.Precision.HIGHEST,
                             preferred_element_type=jnp.float32)
    c = jax.nn.relu(jnp.dot(pooled, fw1_ref[...], preferred_element_type=jnp.float32)
                    + fb1_ref[...])
    c = jax.nn.relu(jnp.dot(c, fw2_ref[...], preferred_element_type=jnp.float32)
                    + fb2_ref[...])
    out_ref[...] = (jnp.dot(c, fw3_ref[...], preferred_element_type=jnp.float32)
                    + fb3_ref[...])


def _tc_fin(parts, y, dinv, b, g, be, batch, fw1, fb1, fw2, fb2, fw3, fb3):
    return pl.pallas_call(
        _tc_fin_body,
        out_shape=jax.ShapeDtypeStruct((G, fw3.shape[1]), jnp.float32),
    )(parts, y, dinv, b.reshape(1, -1), g.reshape(1, -1), be.reshape(1, -1),
      batch.reshape(N, 1), fw1, fb1.reshape(1, -1), fw2, fb2.reshape(1, -1),
      fw3, fb3.reshape(1, -1))


def kernel(x, edge_index, edge_w, batch, W1, b1, g1, be1, W2, b2, g2, be2,
           W3, b3, g3, be3, fw1, fb1, fw2, fb2, fw3, fb3):
    row = edge_index[0]
    col = edge_index[1]
    pad = EP - E
    zi = jnp.zeros((pad,), jnp.int32)
    row_p = jnp.concatenate([row, zi]).reshape(NW, NCHUNK, CH)
    col_p = jnp.concatenate([col, zi]).reshape(NW, NCHUNK, CH)
    ew_p = jnp.concatenate([edge_w, jnp.zeros((pad,), jnp.float32)]).reshape(NW, NCHUNK, CH)

    ew_p16 = ew_p.reshape(NW, EPT // 16, 16)
    row_p160 = row_p.reshape(NW, EPT // 160, 160)
    col_p160 = col_p.reshape(NW, EPT // 160, 160)
    deg_parts = _deg_call(col_p, ew_p).reshape(2, NP, 1)
    dinv, y1 = _tc1(deg_parts, x, W1)
    p1 = _agg_128(row_p160, col_p160, ew_p16, y1)
    y2 = _tc_mid(p1, y1, dinv, b1, g1, be1, W2)
    p2 = _agg_64(row_p160, col_p160, ew_p16, y2)
    y3 = _tc_mid(p2, y2, dinv, b2, g2, be2, W3)
    p3 = _agg_32(row_p160, col_p160, ew_p16, y3)
    return _tc_fin(p3, y3, dinv, b3, g3, be3, batch, fw1, fb1, fw2, fb2, fw3, fb3)


# speedup vs baseline: 1.0139x; 1.0139x over previous
import functools

import jax
import jax.numpy as jnp
from jax import lax
from jax.experimental import pallas as pl
from jax.experimental.pallas import tpu as pltpu
from jax.experimental.pallas import tpu_sc as plsc

N = 10000
E = 160000
G = 64
NP = 10240
ROWS_PER_TILE = NP // 16
NW = 32
CH = 128
NCHUNK = 40
EPT = CH * NCHUNK
EP = NW * EPT

_mesh = plsc.VectorSubcoreMesh(core_axis_name="c", subcore_axis_name="s")


def _zero_rows(zbuf, n_rows, d):
    def body(i, _):
        for k in range(d // 16):
            zbuf[i, pl.ds(16 * k, 16)] = jnp.zeros((16,), jnp.float32)
        return 0
    lax.fori_loop(0, n_rows, body, 0)


def _deg_body(col_hbm, ew_hbm, out_hbm, col_v, ew_v, zbuf, deg_sh):
    cid = lax.axis_index("c")
    sid = lax.axis_index("s")
    wid = cid * 16 + sid
    pltpu.sync_copy(col_hbm.at[wid], col_v)
    pltpu.sync_copy(ew_hbm.at[wid], ew_v)

    def zb(i, _):
        zbuf[pl.ds(16 * i, 16)] = jnp.zeros((16,), jnp.float32)
        return 0
    lax.fori_loop(0, ROWS_PER_TILE // 16, zb, 0)
    pltpu.sync_copy(zbuf, deg_sh.at[pl.ds(sid * ROWS_PER_TILE, ROWS_PER_TILE)])
    plsc.subcore_barrier()

    def chunk(j, _):
        pltpu.sync_copy(ew_v.at[j], deg_sh.at[col_v.at[j]], add=True)
        return 0
    lax.fori_loop(0, NCHUNK, chunk, 0)
    plsc.subcore_barrier()
    pltpu.sync_copy(deg_sh.at[pl.ds(sid * ROWS_PER_TILE, ROWS_PER_TILE)],
                    out_hbm.at[cid, pl.ds(sid * ROWS_PER_TILE, ROWS_PER_TILE)])


@functools.partial(
    pl.kernel,
    out_type=jax.ShapeDtypeStruct((2, NP), jnp.float32),
    mesh=_mesh,
    scratch_types=[
        pltpu.VMEM((NCHUNK, CH), jnp.int32),
        pltpu.VMEM((NCHUNK, CH), jnp.float32),
        pltpu.VMEM((ROWS_PER_TILE,), jnp.float32),
        pltpu.VMEM_SHARED((NP,), jnp.float32),
    ],
)
def _deg_call(col_hbm, ew_hbm, out_hbm, col_v, ew_v, zbuf, deg_sh):
    _deg_body(col_hbm, ew_hbm, out_hbm, col_v, ew_v, zbuf, deg_sh)


NBUF = 2
NROW_TILE = N // 16


def _agg_body(d, ch, nchunk, row_hbm, col_hbm, ew_hbm, y_hbm, out_hbm,
              row_v, col_v, ew_v, gbufs, acc_sh, gsems):
    cid = lax.axis_index("c")
    sid = lax.axis_index("s")
    wid = cid * 16 + sid
    pltpu.sync_copy(row_hbm.at[wid], row_v)
    pltpu.sync_copy(col_hbm.at[wid], col_v)
    pltpu.sync_copy(ew_hbm.at[wid], ew_v)
    _zero_rows(gbufs[0], ch, d)
    for k in range(NROW_TILE // ch):
        pltpu.sync_copy(gbufs[0], acc_sh.at[pl.ds(sid * NROW_TILE + k * ch, ch)])
    rem = NROW_TILE % ch
    if rem:
        pltpu.sync_copy(
            gbufs[0].at[pl.ds(0, rem)],
            acc_sh.at[pl.ds(sid * NROW_TILE + (NROW_TILE // ch) * ch, rem)])
    plsc.subcore_barrier()

    def scale_chunk(j, gbuf):
        for g in range(ch // 16):
            ew16 = ew_v[j * (ch // 16) + g]
            for r in range(16):
                s = lax.gather(
                    ew16, jnp.full((16, 1), r, jnp.int32),
                    lax.GatherDimensionNumbers(
                        offset_dims=(), collapsed_slice_dims=(0,),
                        start_index_map=(0,)),
                    (1,), mode=lax.GatherScatterMode.PROMISE_IN_BOUNDS)
                e = g * 16 + r
                for k in range(d // 16):
                    gbuf[e, pl.ds(16 * k, 16)] = gbuf[e, pl.ds(16 * k, 16)] * s

    def outer(j, _):
        pltpu.async_copy(y_hbm.at[row_v.at[j]], gbufs[0], gsems[0]).wait()
        scale_chunk(j, gbufs[0])
        pltpu.sync_copy(gbufs[0], acc_sh.at[col_v.at[j]], add=True)
        return 0
    lax.fori_loop(0, nchunk, outer, 0)
    plsc.subcore_barrier()
    pltpu.sync_copy(acc_sh.at[pl.ds(sid * NROW_TILE, NROW_TILE)],
                    out_hbm.at[cid, pl.ds(sid * NROW_TILE, NROW_TILE)])


def _make_agg(d, ch):
    nchunk = EPT // ch
    @functools.partial(
        pl.kernel,
        out_type=jax.ShapeDtypeStruct((2, N, d), jnp.float32),
        mesh=_mesh,
        scratch_types=[
            pltpu.VMEM((nchunk, ch), jnp.int32),
            pltpu.VMEM((nchunk, ch), jnp.int32),
            pltpu.VMEM((EPT // 16, 16), jnp.float32),
        ] + [pltpu.VMEM((ch, d), jnp.float32)] * 1 + [
            pltpu.VMEM_SHARED((N, d), jnp.float32),
        ] + [pltpu.SemaphoreType.DMA] * 1,
        compiler_params=pltpu.CompilerParams(use_tc_tiling_on_sc=False),
        name=f"gcn_agg_d{d}",
    )
    def agg(row_hbm, col_hbm, ew_hbm, y_hbm, out_hbm,
            row_v, col_v, ew_v, g0, acc_sh, gs0):
        _agg_body(d, ch, nchunk, row_hbm, col_hbm, ew_hbm, y_hbm, out_hbm,
                  row_v, col_v, ew_v, (g0,), acc_sh, (gs0,))
    return agg


_agg_128 = _make_agg(128, 160)
_agg_64 = _make_agg(64, 256)
_agg_32 = _make_agg(32, 256)


def _tc1_body(parts_ref, x_ref, w_ref, dinv_ref, y_ref):
    p = parts_ref[...]
    deg = p[0, :N] + p[1, :N] + 1.0
    dinv = jnp.where(deg > 0, lax.rsqrt(deg), 0.0)
    t = jnp.dot(x_ref[...], w_ref[...], preferred_element_type=jnp.float32)
    dinv_ref[...] = dinv
    y_ref[...] = dinv * t


def _tc1(parts, x, w):
    return pl.pallas_call(
        _tc1_body,
        out_shape=(jax.ShapeDtypeStruct((N, 1), jnp.float32),
                   jax.ShapeDtypeStruct((N, w.shape[1]), jnp.float32)),
    )(parts, x, w)


def _tc_mid_body(parts_ref, y_ref, dinv_ref, b_ref, g_ref, be_ref, w_ref, ynext_ref):
    p = parts_ref[...]
    agg = p[0, :N] + p[1, :N]
    dinv = dinv_ref[...]
    z = jax.nn.relu(dinv * (agg + y_ref[...]) + b_ref[...])
    m = jnp.mean(z, axis=0, keepdims=True)
    v = jnp.mean((z - m) ** 2, axis=0, keepdims=True)
    h = g_ref[...] * (z - m) * lax.rsqrt(v + 1e-5) + be_ref[...]
    ynext_ref[...] = dinv * jnp.dot(h, w_ref[...], preferred_element_type=jnp.float32)


def _tc_mid(parts, y, dinv, b, g, be, w):
    return pl.pallas_call(
        _tc_mid_body,
        out_shape=jax.ShapeDtypeStruct((N, w.shape[1]), jnp.float32),
    )(parts, y, dinv, b.reshape(1, -1), g.reshape(1, -1), be.reshape(1, -1), w)


def _tc_fin_body(parts_ref, y_ref, dinv_ref, b_ref, g_ref, be_ref, batch_ref,
                 fw1_ref, fb1_ref, fw2_ref, fb2_ref, fw3_ref, fb3_ref, out_ref):
    p = parts_ref[...]
    agg = p[0, :N] + p[1, :N]
    dinv = dinv_ref[...]
    z = jax.nn.relu(dinv * (agg + y_ref[...]) + b_ref[...])
    m = jnp.mean(z, axis=0, keepdims=True)
    v = jnp.mean((z - m) ** 2, axis=0, keepdims=True)
    h = g_ref[...] * (z - m) * lax.rsqrt(v + 1e-5) + be_ref[...]
    oh = (batch_ref[...] == lax.broadcasted_iota(jnp.int32, (N, G), 1))
    pooled = lax.dot_general(oh.astype(jnp.float32), h,
                             (((0,), (0,)), ((), ())),
                             precision=lax.Precision.HIGHEST,
                             preferred_element_type=jnp.float32)
    c = jax.nn.relu(jnp.dot(pooled, fw1_ref[...], preferred_element_type=jnp.float32)
                    + fb1_ref[...])
    c = jax.nn.relu(jnp.dot(c, fw2_ref[...], preferred_element_type=jnp.float32)
                    + fb2_ref[...])
    out_ref[...] = (jnp.dot(c, fw3_ref[...], preferred_element_type=jnp.float32)
                    + fb3_ref[...])


def _tc_fin(parts, y, dinv, b, g, be, batch, fw1, fb1, fw2, fb2, fw3, fb3):
    return pl.pallas_call(
        _tc_fin_body,
        out_shape=jax.ShapeDtypeStruct((G, fw3.shape[1]), jnp.float32),
    )(parts, y, dinv, b.reshape(1, -1), g.reshape(1, -1), be.reshape(1, -1),
      batch.reshape(N, 1), fw1, fb1.reshape(1, -1), fw2, fb2.reshape(1, -1),
      fw3, fb3.reshape(1, -1))


def kernel(x, edge_index, edge_w, batch, W1, b1, g1, be1, W2, b2, g2, be2,
           W3, b3, g3, be3, fw1, fb1, fw2, fb2, fw3, fb3):
    row = edge_index[0]
    col = edge_index[1]
    pad = EP - E
    zi = jnp.zeros((pad,), jnp.int32)
    row_p = jnp.concatenate([row, zi]).reshape(NW, NCHUNK, CH)
    col_p = jnp.concatenate([col, zi]).reshape(NW, NCHUNK, CH)
    ew_p = jnp.concatenate([edge_w, jnp.zeros((pad,), jnp.float32)]).reshape(NW, NCHUNK, CH)

    ew_p16 = ew_p.reshape(NW, EPT // 16, 16)
    row_p160 = row_p.reshape(NW, EPT // 160, 160)
    col_p160 = col_p.reshape(NW, EPT // 160, 160)
    row_p256 = row_p.reshape(NW, EPT // 256, 256)
    col_p256 = col_p.reshape(NW, EPT // 256, 256)
    deg_parts = _deg_call(col_p, ew_p).reshape(2, NP, 1)
    dinv, y1 = _tc1(deg_parts, x, W1)
    p1 = _agg_128(row_p160, col_p160, ew_p16, y1)
    y2 = _tc_mid(p1, y1, dinv, b1, g1, be1, W2)
    p2 = _agg_64(row_p256, col_p256, ew_p16, y2)
    y3 = _tc_mid(p2, y2, dinv, b2, g2, be2, W3)
    p3 = _agg_32(row_p256, col_p256, ew_p16, y3)
    return _tc_fin(p3, y3, dinv, b3, g3, be3, batch, fw1, fb1, fw2, fb2, fw3, fb3)
